# baseline (speedup 1.0000x reference)
import functools

import numpy as np
import jax
import jax.numpy as jnp
from jax import lax
from jax.experimental import pallas as pl
from jax.experimental.pallas import tpu as pltpu
from jax.experimental.pallas import tpu_sc as plsc

_B, _L, _D = 4, 4096, 1024
_NC, _NS = 2, 16
_NW = _NC * _NS
_ROWS = _B * _L
_RPW = _ROWS // _NW
_CHUNK = 32
_NCHUNK = _RPW // _CHUNK
_NBUF = 3


def _flat_forward_perm_indices() -> np.ndarray:
    mseq = np.arange(_L)
    perm = np.zeros((_B, _L), dtype=np.int64)
    for i in range(_B):
        np.random.seed(i)
        perm[i] = np.random.permutation(mseq)
    flat = perm + (np.arange(_B)[:, None] * _L)
    return flat.reshape(_NW, _NCHUNK, _CHUNK).astype(np.int32)


_FLAT_IDX = _flat_forward_perm_indices()

_mesh = plsc.VectorSubcoreMesh(core_axis_name="c", subcore_axis_name="s")


@functools.partial(
    pl.kernel,
    mesh=_mesh,
    out_type=jax.ShapeDtypeStruct((_ROWS, _D), jnp.float32),
    scratch_types=[
        pltpu.VMEM((_NCHUNK, _CHUNK), jnp.int32),
        pltpu.VMEM((_NBUF, _CHUNK, _D), jnp.float32),
    ]
    + [pltpu.SemaphoreType.DMA] * (2 * _NBUF),
)
def _deinterleave(x_hbm, idx_hbm, out_hbm, idx_v, rows_v, *sems):
    sg, sw = sems[:_NBUF], sems[_NBUF:]
    wid = lax.axis_index("s") * _NC + lax.axis_index("c")
    base = wid * _RPW
    pltpu.sync_copy(idx_hbm.at[wid], idx_v)

    def fire_read(k):
        return pltpu.async_copy(
            x_hbm.at[pl.ds(base + k * _CHUNK, _CHUNK)], rows_v.at[k % _NBUF],
            sg[k % _NBUF])

    reads = [fire_read(k) for k in range(_NBUF - 1)]
    writes = [None] * _NCHUNK
    for j in range(_NCHUNK):
        k = j + _NBUF - 1
        if k < _NCHUNK:
            if j >= 1:
                writes[j - 1].wait()
                writes[j - 1] = None
            reads.append(fire_read(k))
        reads[j].wait()
        writes[j] = pltpu.async_copy(
            rows_v.at[j % _NBUF], out_hbm.at[idx_v.at[j]], sw[j % _NBUF])
    for w in writes:
        if w is not None:
            w.wait()


def kernel(x):
    x2 = x.reshape(_ROWS, _D)
    idx = jnp.asarray(_FLAT_IDX)
    y = _deinterleave(x2, idx)
    return y.reshape(_B, _L, _D)


# speedup vs baseline: 1.0142x; 1.0142x over previous
import functools

import numpy as np
import jax
import jax.numpy as jnp
from jax import lax
from jax.experimental import pallas as pl
from jax.experimental.pallas import tpu as pltpu
from jax.experimental.pallas import tpu_sc as plsc

_B, _L, _D = 4, 4096, 1024
_NC, _NS = 2, 16
_NW = _NC * _NS
_ROWS = _B * _L
_RPW = _ROWS // _NW
_CHUNK = 16
_NCHUNK = _RPW // _CHUNK
_NBUF = 4


def _flat_inverse_perm_indices() -> np.ndarray:
    mseq = np.arange(_L)
    idx = np.zeros((_B, _L), dtype=np.int64)
    for i in range(_B):
        np.random.seed(i)
        mshuf = np.random.permutation(mseq)
        idx[i] = np.argsort(mshuf)
    flat = idx + (np.arange(_B)[:, None] * _L)
    return flat.reshape(_NW, _NCHUNK, _CHUNK).astype(np.int32)


_FLAT_IDX = _flat_inverse_perm_indices()

_mesh = plsc.VectorSubcoreMesh(core_axis_name="c", subcore_axis_name="s")


@functools.partial(
    pl.kernel,
    mesh=_mesh,
    out_type=jax.ShapeDtypeStruct((_ROWS, _D), jnp.float32),
    scratch_types=[
        pltpu.VMEM((_NCHUNK, _CHUNK), jnp.int32),
        pltpu.VMEM((_NBUF, _CHUNK, _D), jnp.float32),
    ]
    + [pltpu.SemaphoreType.DMA] * (2 * _NBUF),
)
def _deinterleave(x_hbm, idx_hbm, out_hbm, idx_v, rows_v, *sems):
    sg, sw = sems[:_NBUF], sems[_NBUF:]
    wid = lax.axis_index("s") * _NC + lax.axis_index("c")
    base = wid * _RPW
    pltpu.sync_copy(idx_hbm.at[wid], idx_v)

    def fire_gather(k):
        return pltpu.async_copy(x_hbm.at[idx_v.at[k]], rows_v.at[k % _NBUF],
                                sg[k % _NBUF])

    gathers = [fire_gather(k) for k in range(_NBUF - 1)]
    writes = [None] * _NCHUNK
    for j in range(_NCHUNK):
        k = j + _NBUF - 1
        if k < _NCHUNK:
            if j >= 1:
                writes[j - 1].wait()
                writes[j - 1] = None
            gathers.append(fire_gather(k))
        gathers[j].wait()
        writes[j] = pltpu.async_copy(
            rows_v.at[j % _NBUF], out_hbm.at[pl.ds(base + j * _CHUNK, _CHUNK)],
            sw[j % _NBUF])
    for w in writes:
        if w is not None:
            w.wait()


def kernel(x):
    x2 = x.reshape(_ROWS, _D)
    idx = jnp.asarray(_FLAT_IDX)
    y = _deinterleave(x2, idx)
    return y.reshape(_B, _L, _D)


# speedup vs baseline: 1.0385x; 1.0240x over previous
import functools

import numpy as np
import jax
import jax.numpy as jnp
from jax import lax
from jax.experimental import pallas as pl
from jax.experimental.pallas import tpu as pltpu
from jax.experimental.pallas import tpu_sc as plsc

_B, _L, _D = 4, 4096, 1024
_NC, _NS = 2, 16
_NW = _NC * _NS
_ROWS = _B * _L
_RPW = _ROWS // _NW
_CHUNK = 32
_NCHUNK = _RPW // _CHUNK
_NBUF = 3


def _flat_inverse_perm_indices() -> np.ndarray:
    mseq = np.arange(_L)
    idx = np.zeros((_B, _L), dtype=np.int64)
    for i in range(_B):
        np.random.seed(i)
        mshuf = np.random.permutation(mseq)
        idx[i] = np.argsort(mshuf)
    flat = idx + (np.arange(_B)[:, None] * _L)
    return flat.reshape(_NW, _NCHUNK, _CHUNK).astype(np.int32)


_FLAT_IDX = _flat_inverse_perm_indices()

_mesh = plsc.VectorSubcoreMesh(core_axis_name="c", subcore_axis_name="s")


@functools.partial(
    pl.kernel,
    mesh=_mesh,
    out_type=jax.ShapeDtypeStruct((_ROWS, _D), jnp.float32),
    scratch_types=[
        pltpu.VMEM((_NCHUNK, _CHUNK), jnp.int32),
        pltpu.VMEM((_NBUF, _CHUNK, _D), jnp.float32),
    ]
    + [pltpu.SemaphoreType.DMA] * (2 * _NBUF),
)
def _deinterleave(x_hbm, idx_hbm, out_hbm, idx_v, rows_v, *sems):
    sg, sw = sems[:_NBUF], sems[_NBUF:]
    wid = lax.axis_index("s") * _NC + lax.axis_index("c")
    base = wid * _RPW
    pltpu.sync_copy(idx_hbm.at[wid], idx_v)

    def gather(j, buf):
        return pltpu.make_async_copy(x_hbm.at[idx_v.at[j]], rows_v.at[buf],
                                     sg[buf])

    def write(j, buf):
        return pltpu.make_async_copy(
            rows_v.at[buf], out_hbm.at[pl.ds(base + j * _CHUNK, _CHUNK)],
            sw[buf])

    for k in range(_NBUF):
        gather(k, k).start()
    gather(0, 0).wait()
    write(0, 0).start()

    @pl.loop(1, _NCHUNK, step=_NBUF)
    def _steady(j0):
        for b in range(_NBUF):
            j = j0 + b
            sj = (1 + b) % _NBUF
            sn = (b + 3) % _NBUF

            @pl.when(j + 2 < _NCHUNK)
            def _():
                write(j - 1, sn).wait()
                gather(j + 2, sn).start()

            gather(j, sj).wait()
            write(j, sj).start()

    for j in range(_NCHUNK - _NBUF, _NCHUNK):
        write(j, j % _NBUF).wait()


def kernel(x):
    x2 = x.reshape(_ROWS, _D)
    idx = jnp.asarray(_FLAT_IDX)
    y = _deinterleave(x2, idx)
    return y.reshape(_B, _L, _D)


# speedup vs baseline: 1.0530x; 1.0140x over previous
import functools

import numpy as np
import jax
import jax.numpy as jnp
from jax import lax
from jax.experimental import pallas as pl
from jax.experimental.pallas import tpu as pltpu
from jax.experimental.pallas import tpu_sc as plsc

_B, _L, _D = 4, 4096, 1024
_NC, _NS = 2, 16
_NW = _NC * _NS
_ROWS = _B * _L
_RPW = _ROWS // _NW
_CHUNK = 16
_NCHUNK = _RPW // _CHUNK
_NBUF = 6
_AHEAD = 4
_PEEL = 2


def _flat_inverse_perm_indices() -> np.ndarray:
    mseq = np.arange(_L)
    idx = np.zeros((_B, _L), dtype=np.int64)
    for i in range(_B):
        np.random.seed(i)
        mshuf = np.random.permutation(mseq)
        idx[i] = np.argsort(mshuf)
    flat = idx + (np.arange(_B)[:, None] * _L)
    return flat.reshape(_NW, _NCHUNK, _CHUNK).astype(np.int32)


_FLAT_IDX = _flat_inverse_perm_indices()

_mesh = plsc.VectorSubcoreMesh(core_axis_name="c", subcore_axis_name="s")


@functools.partial(
    pl.kernel,
    mesh=_mesh,
    out_type=jax.ShapeDtypeStruct((_ROWS, _D), jnp.float32),
    scratch_types=[
        pltpu.VMEM((_NCHUNK, _CHUNK), jnp.int32),
        pltpu.VMEM((_NBUF, _CHUNK, _D), jnp.float32),
    ]
    + [pltpu.SemaphoreType.DMA] * (2 * _NBUF),
)
def _deinterleave(x_hbm, idx_hbm, out_hbm, idx_v, rows_v, *sems):
    sg, sw = sems[:_NBUF], sems[_NBUF:]
    wid = lax.axis_index("s") * _NC + lax.axis_index("c")
    base = wid * _RPW
    pltpu.sync_copy(idx_hbm.at[wid], idx_v)

    def gather(j, buf):
        return pltpu.make_async_copy(x_hbm.at[idx_v.at[j]], rows_v.at[buf],
                                     sg[buf])

    def write(j, buf):
        return pltpu.make_async_copy(
            rows_v.at[buf], out_hbm.at[pl.ds(base + j * _CHUNK, _CHUNK)],
            sw[buf])

    for k in range(_AHEAD):
        gather(k, k % _NBUF).start()
    for j in range(_PEEL):
        gather(j + _AHEAD, (j + _AHEAD) % _NBUF).start()
        gather(j, j % _NBUF).wait()
        write(j, j % _NBUF).start()

    @pl.loop(_PEEL, _NCHUNK, step=_NBUF)
    def _steady(j0):
        for b in range(_NBUF):
            j = j0 + b
            sj = (_PEEL + b) % _NBUF
            sn = (_PEEL + b - 2) % _NBUF

            @pl.when(j + _AHEAD < _NCHUNK)
            def _():
                write(j - 2, sn).wait()
                gather(j + _AHEAD, sn).start()

            gather(j, sj).wait()
            write(j, sj).start()

    for j in range(_NCHUNK - _AHEAD - 2, _NCHUNK):
        write(j, j % _NBUF).wait()


def kernel(x):
    x2 = x.reshape(_ROWS, _D)
    idx = jnp.asarray(_FLAT_IDX)
    y = _deinterleave(x2, idx)
    return y.reshape(_B, _L, _D)
